# baseline (speedup 1.0000x reference)
import jax
import jax.numpy as jnp
from jax import lax
from jax.experimental import pallas as pl
from jax.experimental.pallas import tpu as pltpu
from jax.experimental.pallas import tpu_sc as plsc

DIM = 64
NWORKERS = 32
W = 400
ROWS_PER_BLOCK = 20000
NARROW_B = 64


def _widen_table(table):
    v = table.shape[0]

    def body(t_ref, o_ref):
        t = t_ref[...]
        o_ref[:, :DIM] = t
        o_ref[:, DIM:] = t

    return pl.pallas_call(
        body,
        grid=(v // ROWS_PER_BLOCK,),
        in_specs=[pl.BlockSpec((ROWS_PER_BLOCK, DIM), lambda i: (i, 0))],
        out_specs=pl.BlockSpec((ROWS_PER_BLOCK, 2 * DIM), lambda i: (i, 0)),
        out_shape=jax.ShapeDtypeStruct((v, 2 * DIM), table.dtype),
        compiler_params=pltpu.CompilerParams(
            dimension_semantics=("parallel",),
        ),
    )(table)


def _narrow_out(wide_out, B, S):
    n = wide_out.shape[0]
    rows = NARROW_B * S

    def body(t_ref, o_ref):
        o_ref[...] = t_ref[:, :DIM]

    out2d = pl.pallas_call(
        body,
        grid=(n // rows,),
        in_specs=[pl.BlockSpec((rows, 2 * DIM), lambda i: (i, 0))],
        out_specs=pl.BlockSpec((rows, DIM), lambda i: (i, 0)),
        out_shape=jax.ShapeDtypeStruct((n, DIM), wide_out.dtype),
        compiler_params=pltpu.CompilerParams(
            dimension_semantics=("parallel",),
        ),
    )(wide_out)
    return out2d.reshape(B, S, DIM)


def kernel(x, table):
    B, S = x.shape
    n = B * S
    idx = x.reshape(n)
    b_per_w = n // NWORKERS
    steps = b_per_w // W

    wide = _widen_table(table)
    mesh = plsc.VectorSubcoreMesh(core_axis_name="c", subcore_axis_name="s")

    @pl.kernel(
        out_type=jax.ShapeDtypeStruct((n, 2 * DIM), table.dtype),
        mesh=mesh,
        scratch_types=[
            pltpu.VMEM((W,), jnp.int32),
            pltpu.VMEM((W, 2 * DIM), jnp.float32),
            pltpu.SemaphoreType.DMA,
        ],
    )
    def gather_kernel(wide_hbm, i_hbm, o_hbm, idx_v, gbuf, sem):
        wid = lax.axis_index("s") * 2 + lax.axis_index("c")

        @pl.loop(0, steps)
        def _(c):
            base = wid * b_per_w + c * W
            pltpu.sync_copy(i_hbm.at[pl.ds(base, W)], idx_v)
            pltpu.async_copy(wide_hbm.at[idx_v], gbuf, sem).wait()
            pltpu.sync_copy(gbuf, o_hbm.at[pl.ds(base, W)])

    out = gather_kernel(wide, idx)
    return _narrow_out(out, B, S)


# speedup vs baseline: 1.0396x; 1.0396x over previous
import jax
import jax.numpy as jnp
from jax import lax
from jax.experimental import pallas as pl
from jax.experimental.pallas import tpu as pltpu
from jax.experimental.pallas import tpu_sc as plsc

DIM = 64
NWORKERS = 32
W = 400
ROWS_PER_BLOCK = 20000
NARROW_B = 64


def _widen_table(table):
    return jnp.concatenate([table, table], axis=1)


def _narrow_out(wide_out, B, S):
    rows = NARROW_B * S

    def body(t_ref, o_ref):
        o_ref[...] = t_ref[:, :DIM].reshape(NARROW_B, S, DIM)

    return pl.pallas_call(
        body,
        grid=(B // NARROW_B,),
        in_specs=[pl.BlockSpec((rows, 2 * DIM), lambda i: (i, 0))],
        out_specs=pl.BlockSpec((NARROW_B, S, DIM), lambda i: (i, 0, 0)),
        out_shape=jax.ShapeDtypeStruct((B, S, DIM), wide_out.dtype),
        compiler_params=pltpu.CompilerParams(
            dimension_semantics=("parallel",),
        ),
    )(wide_out)


def kernel(x, table):
    B, S = x.shape
    n = B * S
    idx = x.reshape(n)
    b_per_w = n // NWORKERS
    steps = b_per_w // W

    wide = _widen_table(table)
    mesh = plsc.VectorSubcoreMesh(core_axis_name="c", subcore_axis_name="s")

    @pl.kernel(
        out_type=jax.ShapeDtypeStruct((n, 2 * DIM), table.dtype),
        mesh=mesh,
        scratch_types=[
            pltpu.VMEM((W,), jnp.int32),
            pltpu.VMEM((W, 2 * DIM), jnp.float32),
            pltpu.SemaphoreType.DMA,
        ],
    )
    def gather_kernel(wide_hbm, i_hbm, o_hbm, idx_v, gbuf, sem):
        wid = lax.axis_index("s") * 2 + lax.axis_index("c")

        @pl.loop(0, steps)
        def _(c):
            base = wid * b_per_w + c * W
            pltpu.sync_copy(i_hbm.at[pl.ds(base, W)], idx_v)
            pltpu.async_copy(wide_hbm.at[idx_v], gbuf, sem).wait()
            pltpu.sync_copy(gbuf, o_hbm.at[pl.ds(base, W)])

    out = gather_kernel(wide, idx)
    return _narrow_out(out, B, S)


# speedup vs baseline: 1.2744x; 1.2259x over previous
import jax
import jax.numpy as jnp
from jax import lax
from jax.experimental import pallas as pl
from jax.experimental.pallas import tpu as pltpu
from jax.experimental.pallas import tpu_sc as plsc

DIM = 64
NWORKERS = 32
W = 800


def kernel(x, table):
    B, S = x.shape
    n = B * S
    idx = x.reshape(n)
    b_per_w = n // NWORKERS
    steps = b_per_w // W

    mesh = plsc.VectorSubcoreMesh(core_axis_name="c", subcore_axis_name="s")
    cp = pltpu.CompilerParams(use_tc_tiling_on_sc=False)

    @pl.kernel(
        out_type=jax.ShapeDtypeStruct((n, DIM), table.dtype),
        mesh=mesh,
        scratch_types=[
            pltpu.VMEM((W,), jnp.int32),
            pltpu.VMEM((W,), jnp.int32),
            pltpu.VMEM((W, DIM), jnp.float32),
            pltpu.VMEM((W, DIM), jnp.float32),
            pltpu.SemaphoreType.DMA,
            pltpu.SemaphoreType.DMA,
            pltpu.SemaphoreType.DMA,
            pltpu.SemaphoreType.DMA,
        ],
        compiler_params=cp,
    )
    def gather_kernel(
        table_hbm, i_hbm, o_hbm, idx0, idx1, g0, g1, si0, si1, sg0, sg1
    ):
        wid = lax.axis_index("s") * 2 + lax.axis_index("c")
        base_w = wid * b_per_w
        o2d = o_hbm
        idxb, gb, sib, sgb = [idx0, idx1], [g0, g1], [si0, si1], [sg0, sg1]

        ih = [None, None]
        gh = [None, None]
        ih[0] = pltpu.async_copy(i_hbm.at[pl.ds(base_w, W)], idxb[0], sib[0])
        for c in range(steps):
            cur = c & 1
            prv = cur ^ 1
            ih[cur].wait()
            gh[cur] = pltpu.async_copy(
                table_hbm.at[idxb[cur]], gb[cur], sgb[cur]
            )
            if c > 0:
                gh[prv].wait()
                pltpu.sync_copy(
                    gb[prv], o2d.at[pl.ds(base_w + (c - 1) * W, W)]
                )
            if c + 1 < steps:
                ih[prv] = pltpu.async_copy(
                    i_hbm.at[pl.ds(base_w + (c + 1) * W, W)], idxb[prv], sib[prv]
                )
        last = (steps - 1) & 1
        gh[last].wait()
        pltpu.sync_copy(gb[last], o2d.at[pl.ds(base_w + (steps - 1) * W, W)])

    return gather_kernel(table, idx).reshape(B, S, DIM)
